# baseline (speedup 1.0000x reference)
import functools
import math

import jax
import jax.numpy as jnp
from jax import lax
from jax.experimental import pallas as pl
from jax.experimental.pallas import tpu as pltpu
from jax.experimental.pallas import tpu_sc as plsc

D_MODEL = 64
SCALE = math.sqrt(D_MODEL)

_info = plsc.get_sparse_core_info()
_NC, _NS, _L = _info.num_cores, _info.num_subcores, _info.num_lanes
_NW = _NC * _NS

_B = 16384 * 50
_B_PER_W = _B // _NW
_CHUNK = 128
_NCHUNK = _B_PER_W // _CHUNK


@functools.partial(
    pl.kernel,
    out_type=jax.ShapeDtypeStruct((_B, D_MODEL), jnp.float32),
    mesh=plsc.VectorSubcoreMesh(core_axis_name="c", subcore_axis_name="s"),
    scratch_types=[
        pltpu.VMEM((_B_PER_W,), jnp.int32),
        pltpu.VMEM((_CHUNK, D_MODEL), jnp.float32),
        pltpu.SemaphoreType.DMA,
        pltpu.SemaphoreType.DMA,
    ],
    compiler_params=pltpu.CompilerParams(use_tc_tiling_on_sc=False),
)
def _emb_kernel(x_hbm, lut_hbm, out_hbm, idx_v, rows_v, in_sem, out_sem):
    wid = lax.axis_index("s") * _NC + lax.axis_index("c")
    base = wid * _B_PER_W
    pltpu.sync_copy(x_hbm.at[pl.ds(base, _B_PER_W)], idx_v)

    @pl.loop(0, _NCHUNK)
    def _chunk(j):
        off = j * _CHUNK
        pltpu.async_copy(
            lut_hbm.at[idx_v.at[pl.ds(off, _CHUNK)]], rows_v, in_sem
        ).wait()

        @pl.loop(0, _CHUNK)
        def _row(r):
            for s in range(D_MODEL // _L):
                sl = pl.ds(s * _L, _L)
                rows_v[r, sl] = rows_v[r, sl] * SCALE

        pltpu.async_copy(
            rows_v, out_hbm.at[pl.ds(base + off, _CHUNK)], out_sem
        ).wait()


def kernel(x, lut):
    x_flat = x.reshape(-1).astype(jnp.int32)
    out = _emb_kernel(x_flat, lut)
    return out.reshape(x.shape[0], x.shape[1], D_MODEL)


# speedup vs baseline: 1.2062x; 1.2062x over previous
import functools
import math

import jax
import jax.numpy as jnp
from jax import lax
from jax.experimental import pallas as pl
from jax.experimental.pallas import tpu as pltpu
from jax.experimental.pallas import tpu_sc as plsc

D_MODEL = 64
SCALE = math.sqrt(D_MODEL)

_info = plsc.get_sparse_core_info()
_NC, _NS, _L = _info.num_cores, _info.num_subcores, _info.num_lanes
_NW = _NC * _NS

_B = 16384 * 50
_B_PER_W = _B // _NW
_CHUNK = 128
_NCHUNK = _B_PER_W // _CHUNK
_NBUF = 4
_NSEG = D_MODEL // _L


@functools.partial(
    pl.kernel,
    out_type=jax.ShapeDtypeStruct((_B, D_MODEL), jnp.float32),
    mesh=plsc.VectorSubcoreMesh(core_axis_name="c", subcore_axis_name="s"),
    scratch_types=[
        pltpu.VMEM((_B_PER_W,), jnp.int32),
        pltpu.VMEM((_NBUF, _CHUNK, D_MODEL), jnp.float32),
        pltpu.VMEM((_NBUF, _CHUNK, D_MODEL), jnp.float32),
    ]
    + [pltpu.SemaphoreType.DMA] * (2 * _NBUF),
    compiler_params=pltpu.CompilerParams(use_tc_tiling_on_sc=False),
)
def _emb_kernel(x_hbm, lut_hbm, out_hbm, idx_v, rin, rout, *sems):
    gsems, ssems = sems[:_NBUF], sems[_NBUF:]
    wid = lax.axis_index("s") * _NC + lax.axis_index("c")
    base = wid * _B_PER_W
    pltpu.sync_copy(x_hbm.at[pl.ds(base, _B_PER_W)], idx_v)

    def fire_gather(j, b):
        pltpu.async_copy(
            lut_hbm.at[idx_v.at[pl.ds(j * _CHUNK, _CHUNK)]], rin.at[b], gsems[b]
        )

    for b in range(_NBUF):
        fire_gather(b, b)

    @pl.loop(0, _NCHUNK // _NBUF)
    def _blk(t):
        for b in range(_NBUF):
            j = t * _NBUF + b
            pltpu.make_async_copy(
                lut_hbm.at[pl.ds(0, _CHUNK)], rin.at[b], gsems[b]
            ).wait()
            @pl.when(t > 0)
            def _drain():
                pltpu.make_async_copy(
                    rout.at[b], out_hbm.at[pl.ds(0, _CHUNK)], ssems[b]
                ).wait()

            @plsc.parallel_loop(0, _CHUNK, unroll=4)
            def _scale(r):
                for s in range(_NSEG):
                    sl = pl.ds(s * _L, _L)
                    rout[b, r, sl] = rin[b, r, sl] * SCALE

            @pl.when(j + _NBUF < _NCHUNK)
            def _next():
                fire_gather(j + _NBUF, b)

            pltpu.async_copy(
                rout.at[b], out_hbm.at[pl.ds(base + j * _CHUNK, _CHUNK)], ssems[b]
            )

    for b in range(_NBUF):
        pltpu.make_async_copy(
            rout.at[b], out_hbm.at[pl.ds(0, _CHUNK)], ssems[b]
        ).wait()


def kernel(x, lut):
    x_flat = x.reshape(-1).astype(jnp.int32)
    out = _emb_kernel(x_flat, lut)
    return out.reshape(x.shape[0], x.shape[1], D_MODEL)
